# baseline (speedup 1.0000x reference)
import functools

import jax
import jax.numpy as jnp
from jax import lax
from jax.experimental import pallas as pl
from jax.experimental.pallas import tpu as pltpu
from jax.experimental.pallas import tpu_sc as plsc

D = 64
BATCH = 4096
HIST = 200
B = BATCH * HIST

NC = 2
NS = 16
NW = NC * NS
PW = B // NW
CHUNK = 128
NCHUNK = PW // CHUNK
LANES = 16


def _sc_body(x_hbm, table_hbm, out_hbm, pidx_v, rows_v, sout_v,
             isem, gsem, osem):
    wid = lax.axis_index("s") * NC + lax.axis_index("c")
    base = wid * PW

    def idx_issue(c, b):
        pltpu.async_copy(
            x_hbm.at[pl.ds(base + c * CHUNK, CHUNK)],
            pidx_v.at[b].at[pl.ds(0, CHUNK)], isem.at[b])

    def idx_wait(b):
        pltpu.make_async_copy(
            x_hbm.at[pl.ds(base, CHUNK)],
            pidx_v.at[b].at[pl.ds(0, CHUNK)], isem.at[b]).wait()

    def pidx_compute(b):
        for k in range(CHUNK // LANES):
            v = pidx_v[b, pl.ds(k * LANES, LANES)]
            pidx_v[b, pl.ds(CHUNK + k * LANES, LANES)] = v & 1
            pidx_v[b, pl.ds(k * LANES, LANES)] = v >> 1

    def gather_issue(b):
        pltpu.async_copy(
            table_hbm.at[pidx_v.at[b].at[pl.ds(0, CHUNK)]], rows_v.at[b],
            gsem.at[b])

    def gather_wait(b):
        pltpu.make_async_copy(
            table_hbm.at[pidx_v.at[b].at[pl.ds(0, CHUNK)]], rows_v.at[b],
            gsem.at[b]).wait()

    def wo_issue(c, b):
        pltpu.async_copy(
            sout_v.at[b], out_hbm.at[pl.ds(base + c * CHUNK, CHUNK)],
            osem.at[b])

    def wo_wait(b):
        pltpu.make_async_copy(
            sout_v.at[b], out_hbm.at[pl.ds(base, CHUNK)], osem.at[b]).wait()

    def compute(b):
        def bbody(k, _):
            hv = pidx_v[b, pl.ds(CHUNK + k * LANES, LANES)] * D
            for r in range(LANES):
                off = hv[r]
                for j in range(D // LANES):
                    v = rows_v[b, k * LANES + r, pl.ds(off + j * LANES, LANES)]
                    sout_v[b, k * LANES + r, pl.ds(j * LANES, LANES)] = (
                        jnp.sign(v))
            return 0
        lax.fori_loop(0, CHUNK // LANES, bbody, 0)

    def chunk_body(c, b, prep_next, wait_wo, stage_next):
        if prep_next:
            b1 = (c + 1) % 2
            idx_wait(b1)
            pidx_compute(b1)
            gather_issue(b1)
        gather_wait(b)
        if wait_wo:
            wo_wait(b)
        compute(b)
        wo_issue(c, b)
        if stage_next:
            idx_issue(c + 2, b)

    idx_issue(0, 0)
    idx_wait(0)
    pidx_compute(0)
    gather_issue(0)
    idx_issue(1, 1)

    chunk_body(0, 0, True, False, True)
    chunk_body(1, 1, True, False, True)

    def outer(t, _):
        c0 = t * 2
        chunk_body(c0, 0, True, True, True)
        chunk_body(c0 + 1, 1, True, True, True)
        return 0

    lax.fori_loop(1, NCHUNK // 2 - 1, outer, 0)

    c0 = NCHUNK - 2
    chunk_body(c0, 0, True, True, False)
    chunk_body(c0 + 1, 1, False, True, False)

    wo_wait(0)
    wo_wait(1)


@functools.partial(jax.jit, static_argnames=())
def kernel(x, table):
    x_flat = x.reshape(-1)
    table2 = table.reshape(-1, 2 * D)
    mesh = plsc.VectorSubcoreMesh(core_axis_name="c", subcore_axis_name="s")
    out = pl.kernel(
        _sc_body,
        mesh=mesh,
        out_type=jax.ShapeDtypeStruct((B, D), jnp.float32),
        scratch_types=[
            pltpu.VMEM((2, 2 * CHUNK), jnp.int32),
            pltpu.VMEM((2, CHUNK, 2 * D), jnp.float32),
            pltpu.VMEM((2, CHUNK, D), jnp.float32),
            pltpu.SemaphoreType.DMA((2,)),
            pltpu.SemaphoreType.DMA((2,)),
            pltpu.SemaphoreType.DMA((2,)),
        ],
    )(x_flat, table2)
    return out.reshape(BATCH, HIST, D)


# speedup vs baseline: 1.3854x; 1.3854x over previous
import functools

import jax
import jax.numpy as jnp
from jax import lax
from jax.experimental import pallas as pl
from jax.experimental.pallas import tpu as pltpu
from jax.experimental.pallas import tpu_sc as plsc

D = 64
BATCH = 4096
HIST = 200
B = BATCH * HIST

NC = 2
NS = 16
NW = NC * NS
PW = B // NW
CHUNK = 128
NCHUNK = PW // CHUNK
LANES = 16


def _sc_body(x_hbm, table_hbm, out_hbm, pidx_v, rows_v, sout_v,
             isem, gsem, osem):
    wid = lax.axis_index("s") * NC + lax.axis_index("c")
    base = wid * PW

    def idx_issue(c, b):
        pltpu.async_copy(
            x_hbm.at[pl.ds(base + c * CHUNK, CHUNK)],
            pidx_v.at[b].at[pl.ds(0, CHUNK)], isem.at[b])

    def idx_wait(b):
        pltpu.make_async_copy(
            x_hbm.at[pl.ds(base, CHUNK)],
            pidx_v.at[b].at[pl.ds(0, CHUNK)], isem.at[b]).wait()

    def gather_issue(b):
        pltpu.async_copy(
            table_hbm.at[pidx_v.at[b].at[pl.ds(0, CHUNK)]], rows_v.at[b],
            gsem.at[b])

    def gather_wait(b):
        pltpu.make_async_copy(
            table_hbm.at[pidx_v.at[b].at[pl.ds(0, CHUNK)]], rows_v.at[b],
            gsem.at[b]).wait()

    def wo_issue(c, b):
        pltpu.async_copy(
            sout_v.at[b], out_hbm.at[pl.ds(base + c * CHUNK, CHUNK)],
            osem.at[b])

    def wo_wait(b):
        pltpu.make_async_copy(
            sout_v.at[b], out_hbm.at[pl.ds(base, CHUNK)], osem.at[b]).wait()

    def compute(b):
        def rbody(i, _):
            for j in range(D // LANES):
                v = rows_v[b, i, pl.ds(j * LANES, LANES)]
                sout_v[b, i, pl.ds(j * LANES, LANES)] = jnp.sign(v)
            return 0
        lax.fori_loop(0, CHUNK, rbody, 0)

    def chunk_body(c, b, prep_next, wait_wo, stage_next):
        if prep_next:
            b1 = (c + 1) % 2
            idx_wait(b1)
            gather_issue(b1)
        gather_wait(b)
        if wait_wo:
            wo_wait(b)
        compute(b)
        wo_issue(c, b)
        if stage_next:
            idx_issue(c + 2, b)

    idx_issue(0, 0)
    idx_wait(0)
    gather_issue(0)
    idx_issue(1, 1)

    chunk_body(0, 0, True, False, True)
    chunk_body(1, 1, True, False, True)

    def outer(t, _):
        c0 = t * 2
        chunk_body(c0, 0, True, True, True)
        chunk_body(c0 + 1, 1, True, True, True)
        return 0

    lax.fori_loop(1, NCHUNK // 2 - 1, outer, 0)

    c0 = NCHUNK - 2
    chunk_body(c0, 0, True, True, False)
    chunk_body(c0 + 1, 1, False, True, False)

    wo_wait(0)
    wo_wait(1)


@functools.partial(jax.jit, static_argnames=())
def kernel(x, table):
    x_flat = x.reshape(-1)
    table2 = jnp.pad(table, ((0, 0), (0, D)))
    mesh = plsc.VectorSubcoreMesh(core_axis_name="c", subcore_axis_name="s")
    out = pl.kernel(
        _sc_body,
        mesh=mesh,
        out_type=jax.ShapeDtypeStruct((B, D), jnp.float32),
        scratch_types=[
            pltpu.VMEM((2, 2 * CHUNK), jnp.int32),
            pltpu.VMEM((2, CHUNK, 2 * D), jnp.float32),
            pltpu.VMEM((2, CHUNK, D), jnp.float32),
            pltpu.SemaphoreType.DMA((2,)),
            pltpu.SemaphoreType.DMA((2,)),
            pltpu.SemaphoreType.DMA((2,)),
        ],
    )(x_flat, table2)
    return out.reshape(BATCH, HIST, D)
